# Initial kernel scaffold; baseline (speedup 1.0000x reference)
import jax
import jax.numpy as jnp
from jax.experimental import pallas as pl


def kernel(x, nextHid, periodHid, qhh, aH, poi_loc, poi_dist, nodes, edge_index, Wi1, Wh1, bi1, bh1, Wi3, Wh3, bi3, bh3, weight, W1, as1, ad1, Wo, aso, ado, user, pre):
    raise NotImplementedError("write your pallas kernel here")


# speedup vs baseline: 115.3650x; 115.3650x over previous
import functools

import jax
import jax.numpy as jnp
from jax import lax
from jax.experimental import pallas as pl
from jax.experimental.pallas import tpu as pltpu
from jax.experimental.pallas import tpu_sc as plsc

N = 100000
E = 1600000
P = 4096
HID = 10
NH = 4
F1 = 4

CHUNK = 512
SUB = 128
NSUB = CHUNK // SUB
NCHUNK = E // CHUNK
NC = 2
NS = 16
NW = NC * NS
KITER = (NCHUNK + NW - 1) // NW
NP = 100352
RPT = NP // NS
RCH = 448
NRC = RPT // RCH

W16 = 16


def _c16(v):
    return jnp.full((16,), v, jnp.int32)


def _sc_l1_body(src3, dst3, sd1, z512, out, sidx, didx, srows, drows, orows,
                acc, sem):
    cid = lax.axis_index("c")
    sid = lax.axis_index("s")
    wid = sid * NC + cid
    r0 = sid * RPT
    lane = lax.broadcasted_iota(jnp.int32, (16,), 0)

    pltpu.sync_copy(z512, orows)
    for t in range(NRC):
        pltpu.sync_copy(orows.at[pl.ds(0, RCH)],
                        acc.at[pl.ds(r0 + t * RCH, RCH)])
    plsc.subcore_barrier()

    def chunk_body(k, carry):
        chunk = k * NW + wid

        @pl.when(chunk < NCHUNK)
        def _():
            pltpu.sync_copy(src3.at[chunk], sidx)
            pltpu.sync_copy(dst3.at[chunk], didx)
            descs = []
            for j in range(NSUB):
                descs.append(pltpu.async_copy(
                    sd1.at[sidx.at[j]], srows.at[pl.ds(j * SUB, SUB)], sem))
            for j in range(NSUB):
                descs.append(pltpu.async_copy(
                    sd1.at[didx.at[j]], drows.at[pl.ds(j * SUB, SUB)], sem))
            for d in descs:
                d.wait()

            def group_body(g, c2):
                rows = g * 16 + lane
                for h in range(2):
                    sdv = plsc.load_gather(drows, [rows, _c16(2 + h)])
                    ssv = plsc.load_gather(srows, [rows, _c16(h)])
                    e = ssv + sdv
                    e = jnp.maximum(e, 0.2 * e)
                    ex = jnp.exp(e)
                    plsc.store_scatter(orows, [rows, _c16(h)], ex)
                    for f in range(F1):
                        w = plsc.load_gather(srows, [rows, _c16(4 + h * F1 + f)])
                        plsc.store_scatter(orows, [rows, _c16(2 + h * F1 + f)],
                                           ex * w)
                return c2

            lax.fori_loop(0, CHUNK // 16, group_body, 0)
            adescs = []
            for j in range(NSUB):
                adescs.append(pltpu.async_copy(
                    orows.at[pl.ds(j * SUB, SUB)], acc.at[didx.at[j]], sem,
                    add=True))
            for d in adescs:
                d.wait()
        return carry

    lax.fori_loop(0, KITER, chunk_body, 0)
    plsc.subcore_barrier()
    for t in range(NRC):
        pltpu.sync_copy(acc.at[pl.ds(r0 + t * RCH, RCH)],
                        orows.at[pl.ds(0, RCH)])
        pltpu.sync_copy(orows.at[pl.ds(0, RCH)],
                        out.at[cid, pl.ds(r0 + t * RCH, RCH)])


def _sc_l2_body(src3, dst3, sd2, z512, out, sidx, didx, srows, drows, orows,
                acc, sem):
    cid = lax.axis_index("c")
    sid = lax.axis_index("s")
    wid = sid * NC + cid
    r0 = sid * RPT
    lane = lax.broadcasted_iota(jnp.int32, (16,), 0)

    pltpu.sync_copy(z512, orows)
    for t in range(NRC):
        pltpu.sync_copy(orows.at[pl.ds(0, RCH)],
                        acc.at[pl.ds(r0 + t * RCH, RCH)])
    plsc.subcore_barrier()

    def chunk_body(k, carry):
        chunk = k * NW + wid

        @pl.when(chunk < NCHUNK)
        def _():
            pltpu.sync_copy(src3.at[chunk], sidx)
            pltpu.sync_copy(dst3.at[chunk], didx)
            descs = []
            for j in range(NSUB):
                descs.append(pltpu.async_copy(
                    sd2.at[sidx.at[j]], srows.at[pl.ds(j * SUB, SUB)], sem))
            for j in range(NSUB):
                descs.append(pltpu.async_copy(
                    sd2.at[didx.at[j]], drows.at[pl.ds(j * SUB, SUB)], sem))
            for d in descs:
                d.wait()

            def group_body(g, c2):
                rows = g * 16 + lane
                sdv = plsc.load_gather(drows, [rows, _c16(1)])
                ssv = plsc.load_gather(srows, [rows, _c16(0)])
                w0 = plsc.load_gather(srows, [rows, _c16(2)])
                w1 = plsc.load_gather(srows, [rows, _c16(3)])
                e = ssv + sdv
                e = jnp.maximum(e, 0.2 * e)
                ex = jnp.exp(e)
                plsc.store_scatter(orows, [rows, _c16(0)], ex)
                plsc.store_scatter(orows, [rows, _c16(1)], ex * w0)
                plsc.store_scatter(orows, [rows, _c16(2)], ex * w1)
                return c2

            lax.fori_loop(0, CHUNK // 16, group_body, 0)
            adescs = []
            for j in range(NSUB):
                adescs.append(pltpu.async_copy(
                    orows.at[pl.ds(j * SUB, SUB)], acc.at[didx.at[j]], sem,
                    add=True))
            for d in adescs:
                d.wait()
        return carry

    lax.fori_loop(0, KITER, chunk_body, 0)
    plsc.subcore_barrier()
    for t in range(NRC):
        pltpu.sync_copy(acc.at[pl.ds(r0 + t * RCH, RCH)],
                        orows.at[pl.ds(0, RCH)])
        pltpu.sync_copy(orows.at[pl.ds(0, RCH)],
                        out.at[cid, pl.ds(r0 + t * RCH, RCH)])


@functools.lru_cache(maxsize=None)
def _sc_kernels():
    mesh = plsc.VectorSubcoreMesh(core_axis_name="c", subcore_axis_name="s",
                                  num_cores=NC, num_subcores=NS)
    params = pltpu.CompilerParams(use_tc_tiling_on_sc=False,
                                  needs_layout_passes=False)
    scratch = [
        pltpu.VMEM((NSUB, SUB), jnp.int32),
        pltpu.VMEM((NSUB, SUB), jnp.int32),
        pltpu.VMEM((CHUNK, W16), jnp.float32),
        pltpu.VMEM((CHUNK, W16), jnp.float32),
        pltpu.VMEM((CHUNK, W16), jnp.float32),
        pltpu.VMEM_SHARED((NP, W16), jnp.float32),
        pltpu.SemaphoreType.DMA,
    ]
    l1 = pl.kernel(
        _sc_l1_body,
        out_type=jax.ShapeDtypeStruct((NC, NP, W16), jnp.float32),
        mesh=mesh, compiler_params=params, scratch_types=scratch)
    l2 = pl.kernel(
        _sc_l2_body,
        out_type=jax.ShapeDtypeStruct((NC, NP, W16), jnp.float32),
        mesh=mesh, compiler_params=params, scratch_types=scratch)
    return l1, l2


def _sc_l1(src3, dst3, sd1, z512):
    return _sc_kernels()[0](src3, dst3, sd1, z512)


def _sc_l2(src3, dst3, sd2, z512):
    return _sc_kernels()[1](src3, dst3, sd2, z512)


def _gru_rows(xr, hprev, WiT, WhT, bi, bh):
    gi = jnp.dot(xr, WiT, preferred_element_type=jnp.float32) + bi
    gh = jnp.dot(hprev, WhT, preferred_element_type=jnp.float32) + bh
    r = jax.nn.sigmoid(gi[:, :HID] + gh[:, :HID])
    z = jax.nn.sigmoid(gi[:, HID:2 * HID] + gh[:, HID:2 * HID])
    n = jnp.tanh(gi[:, 2 * HID:] + r * gh[:, 2 * HID:])
    return (1.0 - z) * n + z * hprev


def _prologue_body(x_ref, nh_ref, x3_ref, phid_ref, qhh_ref, aH_ref,
                   pld_ref, pd_ref, wi1_ref, wh1_ref, bi1_ref, bh1_ref,
                   wi3_ref, wh3_ref, bi3_ref, bh3_ref, w_ref,
                   h_ref, ph_ref, qhh2_ref, aH2_ref, ms_ref):
    h = _gru_rows(x_ref[...], nh_ref[...], wi1_ref[...], wh1_ref[...],
                  bi1_ref[...], bh1_ref[...])
    ph = _gru_rows(x3_ref[...], phid_ref[...], wi3_ref[...], wh3_ref[...],
                   bi3_ref[...], bh3_ref[...])
    h_ref[...] = h
    ph_ref[...] = ph

    s = w_ref[0, 0] * h
    dfac = jnp.exp(-pd_ref[...] / 2.0)
    pl0 = pld_ref[...][:, 0:1]
    pl1 = pld_ref[...][:, 1:2]
    E0 = jnp.exp(pl0 * dfac * s)
    E1 = jnp.exp(pl1 * dfac * s)
    r0 = (jnp.sum(E0 * pl0, axis=0, keepdims=True)
          / jnp.sum(E0, axis=0, keepdims=True))
    r1 = (jnp.sum(E1 * pl1, axis=0, keepdims=True)
          / jnp.sum(E1, axis=0, keepdims=True))

    qhi = jnp.exp(lax.dot_general(h, ph, (((0,), (0,)), ((), ())),
                                  preferred_element_type=jnp.float32))
    qhh2 = qhh_ref[...] + qhi
    aH2 = aH_ref[...] + qhi / qhh2
    cP = aH2 * ph
    qhh2_ref[...] = qhh2
    aH2_ref[...] = aH2

    mx = jnp.maximum(jnp.maximum(jnp.max(h), jnp.max(cP)),
                     jnp.maximum(jnp.max(r0), jnp.max(r1)))
    mn = jnp.minimum(jnp.minimum(jnp.min(h), jnp.min(cP)),
                     jnp.minimum(jnp.min(r0), jnp.min(r1)))
    ms_ref[...] = jnp.concatenate([mx.reshape(1, 1), mn.reshape(1, 1)],
                                  axis=1)


def _prologue(x, nextHid, x3, periodHid, qhh, aH, pld, pd,
              Wi1T, Wh1T, bi1r, bh1r, Wi3T, Wh3T, bi3r, bh3r, weight):
    f32 = jnp.float32
    return pl.pallas_call(
        _prologue_body,
        out_shape=(
            jax.ShapeDtypeStruct((1, HID), f32),
            jax.ShapeDtypeStruct((1, HID), f32),
            jax.ShapeDtypeStruct((HID, HID), f32),
            jax.ShapeDtypeStruct((HID, HID), f32),
            jax.ShapeDtypeStruct((1, 2), f32),
        ),
    )(x, nextHid, x3, periodHid, qhh, aH, pld, pd,
      Wi1T, Wh1T, bi1r, bh1r, Wi3T, Wh3T, bi3r, bh3r, weight)


_BLK = 1000


def _nodes_pre_body(nodes_ref, w1c_ref, as_ref, ad_ref, za_ref, zb_ref):
    Wh = jnp.dot(nodes_ref[...], w1c_ref[...],
                 preferred_element_type=jnp.float32)
    ss = jnp.dot(Wh, as_ref[...], preferred_element_type=jnp.float32)
    sd = jnp.dot(Wh, ad_ref[...], preferred_element_type=jnp.float32)
    pad = jnp.zeros((Wh.shape[0], 4), jnp.float32)
    za_ref[...] = jnp.concatenate(
        [ss[:, 0:2], sd[:, 0:2], Wh[:, 0:8], pad], axis=1)
    zb_ref[...] = jnp.concatenate(
        [ss[:, 2:4], sd[:, 2:4], Wh[:, 8:16], pad], axis=1)


def _nodes_pre(nodes, W1cat, AS, AD):
    f32 = jnp.float32
    return pl.pallas_call(
        _nodes_pre_body,
        grid=(N // _BLK,),
        in_specs=[
            pl.BlockSpec((_BLK, F1), lambda i: (i, 0)),
            pl.BlockSpec((F1, NH * F1), lambda i: (0, 0)),
            pl.BlockSpec((NH * F1, NH), lambda i: (0, 0)),
            pl.BlockSpec((NH * F1, NH), lambda i: (0, 0)),
        ],
        out_specs=(
            pl.BlockSpec((_BLK, W16), lambda i: (i, 0)),
            pl.BlockSpec((_BLK, W16), lambda i: (i, 0)),
        ),
        out_shape=(
            jax.ShapeDtypeStruct((N, W16), f32),
            jax.ShapeDtypeStruct((N, W16), f32),
        ),
    )(nodes, W1cat, AS, AD)


def _layer2_pre_body(a0_ref, a1_ref, b0_ref, b1_ref, m2_ref, s2_ref):
    A = a0_ref[0] + a1_ref[0]
    B = b0_ref[0] + b1_ref[0]
    parts = []
    for h in range(2):
        parts.append(A[:, 2 + F1 * h:2 + F1 * h + F1]
                     / (A[:, h:h + 1] + 1e-16))
    for h in range(2):
        parts.append(B[:, 2 + F1 * h:2 + F1 * h + F1]
                     / (B[:, h:h + 1] + 1e-16))
    out1 = jnp.concatenate(parts, axis=1)
    hcat = jnp.where(out1 > 0, out1, jnp.exp(out1) - 1.0)
    s2 = jnp.dot(hcat, m2_ref[...], preferred_element_type=jnp.float32)
    s2_ref[...] = jnp.concatenate(
        [s2, jnp.zeros((s2.shape[0], W16 - 4), jnp.float32)], axis=1)


def _layer2_pre(acc1a, acc1b, M2):
    return pl.pallas_call(
        _layer2_pre_body,
        grid=(N // _BLK,),
        in_specs=[
            pl.BlockSpec((1, _BLK, W16), lambda i: (0, i, 0)),
            pl.BlockSpec((1, _BLK, W16), lambda i: (1, i, 0)),
            pl.BlockSpec((1, _BLK, W16), lambda i: (0, i, 0)),
            pl.BlockSpec((1, _BLK, W16), lambda i: (1, i, 0)),
            pl.BlockSpec((NH * F1, 4), lambda i: (0, 0)),
        ],
        out_specs=pl.BlockSpec((_BLK, W16), lambda i: (i, 0)),
        out_shape=jax.ShapeDtypeStruct((N, W16), jnp.float32),
    )(acc1a, acc1a, acc1b, acc1b, M2)


def _epilogue_body(a_ref, pld_ref, ms_ref, out_ref):
    a = a_ref[0] + a_ref[1]
    den = a[:, 0:1] + 1e-16
    g0 = a[:, 1:2] / den
    g1 = a[:, 2:3] / den
    g0 = jnp.where(g0 > 0, g0, jnp.exp(g0) - 1.0)
    g1 = jnp.where(g1 > 0, g1, jnp.exp(g1) - 1.0)
    m = jnp.maximum(g0, g1)
    ls = m + jnp.log(jnp.exp(g0 - m) + jnp.exp(g1 - m))
    l0 = g0 - ls
    l1 = g1 - ls
    pr0 = l0 * pld_ref[...][:, 0:1]
    pr1 = l1 * pld_ref[...][:, 1:2]
    mlmax = ms_ref[0, 0]
    mlmin = ms_ref[0, 1]
    f0 = jnp.maximum(pr0 * mlmax, pr0 * mlmin)
    f1 = jnp.maximum(pr1 * mlmax, pr1 * mlmin)
    anw = jnp.maximum(f0, f1)
    mval = jnp.max(anw)
    iota = lax.broadcasted_iota(jnp.int32, (P, 1), 0)
    cand = jnp.where(anw == mval, iota, jnp.int32(2 ** 30))
    out_ref[0, 0] = jnp.min(cand)


def _epilogue(sel2, poi_loc, mstat):
    return pl.pallas_call(
        _epilogue_body,
        in_specs=[
            pl.BlockSpec(memory_space=pltpu.VMEM),
            pl.BlockSpec(memory_space=pltpu.VMEM),
            pl.BlockSpec(memory_space=pltpu.SMEM),
        ],
        out_specs=pl.BlockSpec(memory_space=pltpu.SMEM),
        out_shape=jax.ShapeDtypeStruct((1, 1), jnp.int32),
    )(sel2, poi_loc, mstat)


def kernel(x, nextHid, periodHid, qhh, aH, poi_loc, poi_dist, nodes,
           edge_index, Wi1, Wh1, bi1, bh1, Wi3, Wh3, bi3, bh3, weight,
           W1, as1, ad1, Wo, aso, ado, user, pre):
    f32 = jnp.float32
    user_f = jnp.asarray(user, dtype=x.dtype).reshape(1, 1)
    x3 = jnp.concatenate([user_f, x], axis=1)

    h, ph, qhh2, aH2, mstat = _prologue(
        x, nextHid, x3, periodHid, qhh, aH, poi_loc,
        poi_dist.reshape(P, 1), Wi1.T, Wh1.T, bi1.reshape(1, -1),
        bh1.reshape(1, -1), Wi3.T, Wh3.T, bi3.reshape(1, -1),
        bh3.reshape(1, -1), weight)

    W1cat = jnp.transpose(W1, (1, 0, 2)).reshape(F1, NH * F1)
    rows = jnp.arange(NH * F1)
    blockmask = (rows[:, None] // F1 == jnp.arange(NH)[None, :])
    AS = jnp.where(blockmask, as1.reshape(-1, 1), 0.0).astype(f32)
    AD = jnp.where(blockmask, ad1.reshape(-1, 1), 0.0).astype(f32)
    SD1a, SD1b = _nodes_pre(nodes, W1cat, AS, AD)

    src3 = edge_index[0].reshape(NCHUNK, NSUB, SUB)
    dst3 = edge_index[1].reshape(NCHUNK, NSUB, SUB)

    z512 = jnp.zeros((CHUNK, W16), f32)
    acc1a = _sc_l1(src3, dst3, SD1a, z512)
    acc1b = _sc_l1(src3, dst3, SD1b, z512)

    M2 = jnp.concatenate(
        [(Wo @ aso)[:, None], (Wo @ ado)[:, None], Wo], axis=1).astype(f32)
    SD2 = _layer2_pre(acc1a, acc1b, M2)

    acc2 = _sc_l2(src3, dst3, SD2, z512)

    sel2 = lax.dynamic_slice(acc2, (0, pre, 0), (NC, P, W16))
    idx = _epilogue(sel2, poi_loc, mstat)
    return h, ph, qhh2, aH2, idx.reshape(())


# speedup vs baseline: 151.8240x; 1.3160x over previous
import functools

import jax
import jax.numpy as jnp
from jax import lax
from jax.experimental import pallas as pl
from jax.experimental.pallas import tpu as pltpu
from jax.experimental.pallas import tpu_sc as plsc

N = 100000
E = 1600000
P = 4096
HID = 10
NH = 4
F1 = 4

CHUNK = 256
SUB = 128
NSUB = CHUNK // SUB
NCHUNK = E // CHUNK
NC = 2
NS = 16
NW = NC * NS
KITER = 2 * ((NCHUNK + 2 * NW - 1) // (2 * NW))
NP = 100352
RPT = NP // NS
RCH = 224
NRC = RPT // RCH

W16 = 16


def _c16(v):
    return jnp.full((16,), v, jnp.int32)


def _edge_pass_body(compute_group):
    def body(eidx4, sd, z, out, eidxb, didxs, srows, drows, orows, acc,
             gsem0, gsem1, asem0, asem1):
        gsems = (gsem0, gsem1)
        asems = (asem0, asem1)
        cid = lax.axis_index("c")
        sid = lax.axis_index("s")
        wid = sid * NC + cid
        r0 = sid * RPT
        lane = lax.broadcasted_iota(jnp.int32, (16,), 0)

        pltpu.sync_copy(z, orows.at[0])
        pltpu.sync_copy(z, orows.at[1])
        for t in range(NRC):
            pltpu.sync_copy(orows.at[0, pl.ds(0, RCH)],
                            acc.at[pl.ds(r0 + t * RCH, RCH)])
        plsc.subcore_barrier()

        def load_idx(p, chunk):
            pltpu.sync_copy(eidx4.at[chunk], eidxb.at[p])

        def issue_gathers(p):
            for j in range(NSUB):
                pltpu.async_copy(sd.at[eidxb.at[p, 0, j]],
                                 srows.at[p, pl.ds(j * SUB, SUB)], gsems[p])
            for j in range(NSUB):
                pltpu.async_copy(sd.at[eidxb.at[p, 1, j]],
                                 drows.at[p, pl.ds(j * SUB, SUB)], gsems[p])

        def wait_gathers(p):
            for j in range(2 * NSUB):
                pltpu.make_async_copy(
                    sd.at[pl.ds(0, SUB)],
                    srows.at[p, pl.ds(0, SUB)], gsems[p]).wait()

        def issue_scatters(p):
            for j in range(NSUB):
                pltpu.async_copy(orows.at[p, pl.ds(j * SUB, SUB)],
                                 acc.at[didxs.at[p, j]], asems[p], add=True)

        def wait_scatters(p):
            for j in range(NSUB):
                pltpu.make_async_copy(
                    orows.at[p, pl.ds(0, SUB)],
                    acc.at[pl.ds(0, SUB)], asems[p]).wait()

        for p in range(2):
            load_idx(p, p * NW + wid)
            issue_gathers(p)

        def phase(p, kk):
            c = (2 * kk + p) * NW + wid

            @pl.when((c >= 2 * NW) & (c - 2 * NW < NCHUNK))
            def _():
                wait_scatters(p)

            @pl.when(c < NCHUNK)
            def _():
                wait_gathers(p)
                for j in range(NSUB):
                    for q in range(SUB // 16):
                        didxs[p, j, pl.ds(q * 16, 16)] = (
                            eidxb[p, 1, j, pl.ds(q * 16, 16)])
                lax.fori_loop(0, CHUNK // 16,
                              lambda g, cr: compute_group(
                                  g, lane, srows.at[p], drows.at[p],
                                  orows.at[p]) or cr, 0)
                issue_scatters(p)

            @pl.when(c + 2 * NW < NCHUNK)
            def _():
                load_idx(p, c + 2 * NW)
                issue_gathers(p)

        def duo(kk, carry):
            phase(0, kk)
            phase(1, kk)
            return carry

        lax.fori_loop(0, KITER // 2, duo, 0)
        for p in range(2):
            @pl.when((2 * (KITER // 2 - 1) + p) * NW + wid < NCHUNK)
            def _():
                wait_scatters(p)
        plsc.subcore_barrier()
        for t in range(NRC):
            pltpu.sync_copy(acc.at[pl.ds(r0 + t * RCH, RCH)],
                            orows.at[0, pl.ds(0, RCH)])
            pltpu.sync_copy(orows.at[0, pl.ds(0, RCH)],
                            out.at[cid, pl.ds(r0 + t * RCH, RCH)])

    return body


def _l1_group(g, lane, srows, drows, orows):
    rows = g * 16 + lane
    for h in range(2):
        sdv = plsc.load_gather(drows, [rows, _c16(2 + h)])
        ssv = plsc.load_gather(srows, [rows, _c16(h)])
        e = ssv + sdv
        e = jnp.maximum(e, 0.2 * e)
        ex = jnp.exp(e)
        plsc.store_scatter(orows, [rows, _c16(h)], ex)
        for f in range(F1):
            w = plsc.load_gather(srows, [rows, _c16(4 + h * F1 + f)])
            plsc.store_scatter(orows, [rows, _c16(2 + h * F1 + f)], ex * w)


def _l2_group(g, lane, srows, drows, orows):
    rows = g * 16 + lane
    sdv = plsc.load_gather(drows, [rows, _c16(1)])
    ssv = plsc.load_gather(srows, [rows, _c16(0)])
    w0 = plsc.load_gather(srows, [rows, _c16(2)])
    w1 = plsc.load_gather(srows, [rows, _c16(3)])
    e = ssv + sdv
    e = jnp.maximum(e, 0.2 * e)
    ex = jnp.exp(e)
    plsc.store_scatter(orows, [rows, _c16(0)], ex)
    plsc.store_scatter(orows, [rows, _c16(1)], ex * w0)
    plsc.store_scatter(orows, [rows, _c16(2)], ex * w1)


@functools.lru_cache(maxsize=None)
def _sc_kernels():
    mesh = plsc.VectorSubcoreMesh(core_axis_name="c", subcore_axis_name="s",
                                  num_cores=NC, num_subcores=NS)
    params = pltpu.CompilerParams(use_tc_tiling_on_sc=False,
                                  needs_layout_passes=False)
    scratch = [
        pltpu.VMEM((2, 2, NSUB, SUB), jnp.int32),
        pltpu.VMEM((2, NSUB, SUB), jnp.int32),
        pltpu.VMEM((2, CHUNK, W16), jnp.float32),
        pltpu.VMEM((2, CHUNK, W16), jnp.float32),
        pltpu.VMEM((2, CHUNK, W16), jnp.float32),
        pltpu.VMEM_SHARED((NP, W16), jnp.float32),
        pltpu.SemaphoreType.DMA,
        pltpu.SemaphoreType.DMA,
        pltpu.SemaphoreType.DMA,
        pltpu.SemaphoreType.DMA,
    ]
    l1 = pl.kernel(
        _edge_pass_body(_l1_group),
        out_type=jax.ShapeDtypeStruct((NC, NP, W16), jnp.float32),
        mesh=mesh, compiler_params=params, scratch_types=scratch)
    l2 = pl.kernel(
        _edge_pass_body(_l2_group),
        out_type=jax.ShapeDtypeStruct((NC, NP, W16), jnp.float32),
        mesh=mesh, compiler_params=params, scratch_types=scratch)
    return l1, l2


def _sc_l1(eidx4, sd1, z):
    return _sc_kernels()[0](eidx4, sd1, z)


def _sc_l2(eidx4, sd2, z):
    return _sc_kernels()[1](eidx4, sd2, z)


def _gru_rows(xr, hprev, WiT, WhT, bi, bh):
    gi = jnp.dot(xr, WiT, preferred_element_type=jnp.float32) + bi
    gh = jnp.dot(hprev, WhT, preferred_element_type=jnp.float32) + bh
    r = jax.nn.sigmoid(gi[:, :HID] + gh[:, :HID])
    z = jax.nn.sigmoid(gi[:, HID:2 * HID] + gh[:, HID:2 * HID])
    n = jnp.tanh(gi[:, 2 * HID:] + r * gh[:, 2 * HID:])
    return (1.0 - z) * n + z * hprev


def _prologue_body(x_ref, nh_ref, x3_ref, phid_ref, qhh_ref, aH_ref,
                   pld_ref, pd_ref, wi1_ref, wh1_ref, bi1_ref, bh1_ref,
                   wi3_ref, wh3_ref, bi3_ref, bh3_ref, w_ref,
                   h_ref, ph_ref, qhh2_ref, aH2_ref, ms_ref):
    h = _gru_rows(x_ref[...], nh_ref[...], wi1_ref[...], wh1_ref[...],
                  bi1_ref[...], bh1_ref[...])
    ph = _gru_rows(x3_ref[...], phid_ref[...], wi3_ref[...], wh3_ref[...],
                   bi3_ref[...], bh3_ref[...])
    h_ref[...] = h
    ph_ref[...] = ph

    s = w_ref[0, 0] * h
    dfac = jnp.exp(-pd_ref[...] / 2.0)
    pl0 = pld_ref[...][:, 0:1]
    pl1 = pld_ref[...][:, 1:2]
    E0 = jnp.exp(pl0 * dfac * s)
    E1 = jnp.exp(pl1 * dfac * s)
    r0 = (jnp.sum(E0 * pl0, axis=0, keepdims=True)
          / jnp.sum(E0, axis=0, keepdims=True))
    r1 = (jnp.sum(E1 * pl1, axis=0, keepdims=True)
          / jnp.sum(E1, axis=0, keepdims=True))

    qhi = jnp.exp(lax.dot_general(h, ph, (((0,), (0,)), ((), ())),
                                  preferred_element_type=jnp.float32))
    qhh2 = qhh_ref[...] + qhi
    aH2 = aH_ref[...] + qhi / qhh2
    cP = aH2 * ph
    qhh2_ref[...] = qhh2
    aH2_ref[...] = aH2

    mx = jnp.maximum(jnp.maximum(jnp.max(h), jnp.max(cP)),
                     jnp.maximum(jnp.max(r0), jnp.max(r1)))
    mn = jnp.minimum(jnp.minimum(jnp.min(h), jnp.min(cP)),
                     jnp.minimum(jnp.min(r0), jnp.min(r1)))
    ms_ref[...] = jnp.concatenate([mx.reshape(1, 1), mn.reshape(1, 1)],
                                  axis=1)


def _prologue(x, nextHid, x3, periodHid, qhh, aH, pld, pd,
              Wi1T, Wh1T, bi1r, bh1r, Wi3T, Wh3T, bi3r, bh3r, weight):
    f32 = jnp.float32
    return pl.pallas_call(
        _prologue_body,
        out_shape=(
            jax.ShapeDtypeStruct((1, HID), f32),
            jax.ShapeDtypeStruct((1, HID), f32),
            jax.ShapeDtypeStruct((HID, HID), f32),
            jax.ShapeDtypeStruct((HID, HID), f32),
            jax.ShapeDtypeStruct((1, 2), f32),
        ),
    )(x, nextHid, x3, periodHid, qhh, aH, pld, pd,
      Wi1T, Wh1T, bi1r, bh1r, Wi3T, Wh3T, bi3r, bh3r, weight)


_BLK = 1000


def _nodes_pre_body(nodes_ref, w1c_ref, as_ref, ad_ref, za_ref, zb_ref):
    Wh = jnp.dot(nodes_ref[...], w1c_ref[...],
                 preferred_element_type=jnp.float32)
    ss = jnp.dot(Wh, as_ref[...], preferred_element_type=jnp.float32)
    sd = jnp.dot(Wh, ad_ref[...], preferred_element_type=jnp.float32)
    pad = jnp.zeros((Wh.shape[0], 4), jnp.float32)
    za_ref[...] = jnp.concatenate(
        [ss[:, 0:2], sd[:, 0:2], Wh[:, 0:8], pad], axis=1)
    zb_ref[...] = jnp.concatenate(
        [ss[:, 2:4], sd[:, 2:4], Wh[:, 8:16], pad], axis=1)


def _nodes_pre(nodes, W1cat, AS, AD):
    f32 = jnp.float32
    return pl.pallas_call(
        _nodes_pre_body,
        grid=(N // _BLK,),
        in_specs=[
            pl.BlockSpec((_BLK, F1), lambda i: (i, 0)),
            pl.BlockSpec((F1, NH * F1), lambda i: (0, 0)),
            pl.BlockSpec((NH * F1, NH), lambda i: (0, 0)),
            pl.BlockSpec((NH * F1, NH), lambda i: (0, 0)),
        ],
        out_specs=(
            pl.BlockSpec((_BLK, W16), lambda i: (i, 0)),
            pl.BlockSpec((_BLK, W16), lambda i: (i, 0)),
        ),
        out_shape=(
            jax.ShapeDtypeStruct((N, W16), f32),
            jax.ShapeDtypeStruct((N, W16), f32),
        ),
    )(nodes, W1cat, AS, AD)


def _layer2_pre_body(a0_ref, a1_ref, b0_ref, b1_ref, m2_ref, s2_ref):
    A = a0_ref[0] + a1_ref[0]
    B = b0_ref[0] + b1_ref[0]
    parts = []
    for h in range(2):
        parts.append(A[:, 2 + F1 * h:2 + F1 * h + F1]
                     / (A[:, h:h + 1] + 1e-16))
    for h in range(2):
        parts.append(B[:, 2 + F1 * h:2 + F1 * h + F1]
                     / (B[:, h:h + 1] + 1e-16))
    out1 = jnp.concatenate(parts, axis=1)
    hcat = jnp.where(out1 > 0, out1, jnp.exp(out1) - 1.0)
    s2 = jnp.dot(hcat, m2_ref[...], preferred_element_type=jnp.float32)
    s2_ref[...] = jnp.concatenate(
        [s2, jnp.zeros((s2.shape[0], W16 - 4), jnp.float32)], axis=1)


def _layer2_pre(acc1a, acc1b, M2):
    return pl.pallas_call(
        _layer2_pre_body,
        grid=(N // _BLK,),
        in_specs=[
            pl.BlockSpec((1, _BLK, W16), lambda i: (0, i, 0)),
            pl.BlockSpec((1, _BLK, W16), lambda i: (1, i, 0)),
            pl.BlockSpec((1, _BLK, W16), lambda i: (0, i, 0)),
            pl.BlockSpec((1, _BLK, W16), lambda i: (1, i, 0)),
            pl.BlockSpec((NH * F1, 4), lambda i: (0, 0)),
        ],
        out_specs=pl.BlockSpec((_BLK, W16), lambda i: (i, 0)),
        out_shape=jax.ShapeDtypeStruct((N, W16), jnp.float32),
    )(acc1a, acc1a, acc1b, acc1b, M2)


def _epilogue_body(a_ref, pld_ref, ms_ref, out_ref):
    a = a_ref[0] + a_ref[1]
    den = a[:, 0:1] + 1e-16
    g0 = a[:, 1:2] / den
    g1 = a[:, 2:3] / den
    g0 = jnp.where(g0 > 0, g0, jnp.exp(g0) - 1.0)
    g1 = jnp.where(g1 > 0, g1, jnp.exp(g1) - 1.0)
    m = jnp.maximum(g0, g1)
    ls = m + jnp.log(jnp.exp(g0 - m) + jnp.exp(g1 - m))
    l0 = g0 - ls
    l1 = g1 - ls
    pr0 = l0 * pld_ref[...][:, 0:1]
    pr1 = l1 * pld_ref[...][:, 1:2]
    mlmax = ms_ref[0, 0]
    mlmin = ms_ref[0, 1]
    f0 = jnp.maximum(pr0 * mlmax, pr0 * mlmin)
    f1 = jnp.maximum(pr1 * mlmax, pr1 * mlmin)
    anw = jnp.maximum(f0, f1)
    mval = jnp.max(anw)
    iota = lax.broadcasted_iota(jnp.int32, (P, 1), 0)
    cand = jnp.where(anw == mval, iota, jnp.int32(2 ** 30))
    out_ref[0, 0] = jnp.min(cand)


def _epilogue(sel2, poi_loc, mstat):
    return pl.pallas_call(
        _epilogue_body,
        in_specs=[
            pl.BlockSpec(memory_space=pltpu.VMEM),
            pl.BlockSpec(memory_space=pltpu.VMEM),
            pl.BlockSpec(memory_space=pltpu.SMEM),
        ],
        out_specs=pl.BlockSpec(memory_space=pltpu.SMEM),
        out_shape=jax.ShapeDtypeStruct((1, 1), jnp.int32),
    )(sel2, poi_loc, mstat)


def kernel(x, nextHid, periodHid, qhh, aH, poi_loc, poi_dist, nodes,
           edge_index, Wi1, Wh1, bi1, bh1, Wi3, Wh3, bi3, bh3, weight,
           W1, as1, ad1, Wo, aso, ado, user, pre):
    f32 = jnp.float32
    user_f = jnp.asarray(user, dtype=x.dtype).reshape(1, 1)
    x3 = jnp.concatenate([user_f, x], axis=1)

    h, ph, qhh2, aH2, mstat = _prologue(
        x, nextHid, x3, periodHid, qhh, aH, poi_loc,
        poi_dist.reshape(P, 1), Wi1.T, Wh1.T, bi1.reshape(1, -1),
        bh1.reshape(1, -1), Wi3.T, Wh3.T, bi3.reshape(1, -1),
        bh3.reshape(1, -1), weight)

    W1cat = jnp.transpose(W1, (1, 0, 2)).reshape(F1, NH * F1)
    rows = jnp.arange(NH * F1)
    blockmask = (rows[:, None] // F1 == jnp.arange(NH)[None, :])
    AS = jnp.where(blockmask, as1.reshape(-1, 1), 0.0).astype(f32)
    AD = jnp.where(blockmask, ad1.reshape(-1, 1), 0.0).astype(f32)
    SD1a, SD1b = _nodes_pre(nodes, W1cat, AS, AD)

    eidx4 = jnp.transpose(edge_index.reshape(2, NCHUNK, NSUB, SUB),
                          (1, 0, 2, 3))

    zc = jnp.zeros((CHUNK, W16), f32)
    acc1a = _sc_l1(eidx4, SD1a, zc)
    acc1b = _sc_l1(eidx4, SD1b, zc)

    M2 = jnp.concatenate(
        [(Wo @ aso)[:, None], (Wo @ ado)[:, None], Wo], axis=1).astype(f32)
    SD2 = _layer2_pre(acc1a, acc1b, M2)

    acc2 = _sc_l2(eidx4, SD2, zc)

    sel2 = lax.dynamic_slice(acc2, (0, pre, 0), (NC, P, W16))
    idx = _epilogue(sel2, poi_loc, mstat)
    return h, ph, qhh2, aH2, idx.reshape(())
